# baseline (speedup 1.0000x reference)
import functools

import jax
import jax.numpy as jnp
from jax import lax
from jax.experimental import pallas as pl
from jax.experimental.pallas import tpu as pltpu
from jax.experimental.pallas import tpu_sc as plsc

_B = 16384
_D = 64
_L = 16

_info = plsc.get_sparse_core_info()
_NC = _info.num_cores
_NS = _info.num_subcores
_NW = _NC * _NS
_BPW = _B // _NW
_CHUNK = 128
_NCHUNK = _BPW // _CHUNK


def _gather_body(idx_hbm, table_hbm, out_hbm, idx_v, rows_v, gsem, osem):
    wid = lax.axis_index("s") * _NC + lax.axis_index("c")
    base = wid * _BPW
    for j in range(_NCHUNK):
        pltpu.sync_copy(idx_hbm.at[pl.ds(base + j * _CHUNK, _CHUNK)], idx_v.at[j])
    for j in range(_NCHUNK):
        for i in range(_CHUNK // _L):
            sl = pl.ds(i * _L, _L)
            idx_v[j, sl] = idx_v[j, sl] + 1
    copies = [
        pltpu.make_async_copy(table_hbm.at[idx_v.at[j]], rows_v.at[j], gsem)
        for j in range(_NCHUNK)
    ]
    for c in copies:
        c.start()
    out_copies = []
    for j, c in enumerate(copies):
        c.wait()
        oc = pltpu.make_async_copy(
            rows_v.at[j], out_hbm.at[pl.ds(base + j * _CHUNK, _CHUNK)], osem)
        oc.start()
        out_copies.append(oc)
    for oc in out_copies:
        oc.wait()


@jax.jit
def kernel(style_idx, embeddings):
    mesh = plsc.VectorSubcoreMesh(core_axis_name="c", subcore_axis_name="s")
    f = functools.partial(
        pl.kernel,
        mesh=mesh,
        out_type=jax.ShapeDtypeStruct((_B, _D), jnp.float32),
        compiler_params=pltpu.CompilerParams(use_tc_tiling_on_sc=False),
        scratch_types=[
            pltpu.VMEM((_NCHUNK, _CHUNK), jnp.int32),
            pltpu.VMEM((_NCHUNK, _CHUNK, _D), jnp.float32),
            pltpu.SemaphoreType.DMA,
            pltpu.SemaphoreType.DMA,
        ],
    )(_gather_body)
    return f(style_idx.astype(jnp.int32), embeddings)


# speedup vs baseline: 2.1468x; 2.1468x over previous
import functools

import jax
import jax.numpy as jnp
from jax import lax
from jax.experimental import pallas as pl
from jax.experimental.pallas import tpu as pltpu
from jax.experimental.pallas import tpu_sc as plsc

_B = 16384
_D = 64
_V = 100001
_L = 16

_info = plsc.get_sparse_core_info()
_NC = _info.num_cores
_NS = _info.num_subcores
_NW = _NC * _NS
_DPW = _D // _NW
_IC = 8192
_NIC = _B // _IC
_UNROLL = 4


def _gather_body(idx_hbm, tab_t_hbm, out_t_hbm, idx_v, row_v, out_v):
    wid = lax.axis_index("s") * _NC + lax.axis_index("c")
    for fd in range(_DPW):
        d = wid * _DPW + fd
        pltpu.sync_copy(tab_t_hbm.at[d], row_v)
        for c in range(_NIC):
            pltpu.sync_copy(idx_hbm.at[pl.ds(c * _IC, _IC)], idx_v)

            def gbody(j, _):
                row_view = row_v.at[pl.ds(0, _V)]
                for u in range(_UNROLL):
                    sl = pl.ds((j * _UNROLL + u) * _L, _L)
                    out_v[sl] = plsc.load_gather(row_view, [idx_v[sl] + 1])
                return _

            lax.fori_loop(0, _IC // (_L * _UNROLL), gbody, 0)
            pltpu.sync_copy(out_v, out_t_hbm.at[d, pl.ds(c * _IC, _IC)])


@jax.jit
def kernel(style_idx, embeddings):
    mesh = plsc.VectorSubcoreMesh(core_axis_name="c", subcore_axis_name="s")
    f = functools.partial(
        pl.kernel,
        mesh=mesh,
        out_type=jax.ShapeDtypeStruct((_D, _B), jnp.float32),
        compiler_params=pltpu.CompilerParams(needs_layout_passes=False),
        scratch_types=[
            pltpu.VMEM((_IC,), jnp.int32),
            pltpu.VMEM((_V,), jnp.float32),
            pltpu.VMEM((_IC,), jnp.float32),
        ],
    )(_gather_body)
    out_t = f(style_idx, embeddings.T)
    return out_t.T
